# baseline (speedup 1.0000x reference)
import jax
import jax.numpy as jnp
from jax import lax
from jax.experimental import pallas as pl
from jax.experimental.pallas import tpu as pltpu
from jax.experimental.pallas import tpu_sc as plsc

D_MODEL = 64
SEQ = 200
BATCH = 4096
LANES = 16
NUM_CORES = 2
NUM_SUBCORES = 16
NW = NUM_CORES * NUM_SUBCORES
TOTAL_ROWS = BATCH * SEQ
ROWS_PER_W = TOTAL_ROWS // NW
CHUNK = 128
N_CHUNKS = ROWS_PER_W // CHUNK
NBUF = 3


def _positional_encoding(length, depth):
    half = depth // 2
    positions = jnp.arange(length, dtype=jnp.float32)[:, None]
    depths = jnp.arange(half, dtype=jnp.float32)[None, :]
    angle_rates = 1.0 / (10000.0 ** depths)
    angle_rads = positions * angle_rates
    return jnp.concatenate([jnp.sin(angle_rads), jnp.cos(angle_rads)], axis=-1)


def _sc_body(table_hbm, idx_hbm, pe_hbm, out_hbm,
             idx_v, pe_v, idx2_v, gath_v, outb_v, gsems, ssems):
    wid = lax.axis_index("s") * NUM_CORES + lax.axis_index("c")
    pltpu.sync_copy(idx_hbm.at[wid], idx_v)
    pltpu.sync_copy(pe_hbm, pe_v)
    out_base = wid * (ROWS_PER_W // 2)

    def prep_chunk(g, p):
        for c in range(0, CHUNK, LANES):
            xv = idx_v[g, pl.ds(c, LANES)]
            idx2_v[p][pl.ds(c, LANES)] = xv >> 1

    def start_gather(p):
        pltpu.async_copy(table_hbm.at[idx2_v[p]], gath_v[p], gsems[p])

    def wait_gather(p):
        pltpu.make_async_copy(table_hbm.at[idx2_v[p]], gath_v[p],
                              gsems[p]).wait()

    def start_store(g, p):
        pltpu.async_copy(
            outb_v[p],
            out_hbm.at[pl.ds(out_base + g * (CHUNK // 2), CHUNK // 2)],
            ssems[p])

    def wait_store(p):
        pltpu.make_async_copy(
            outb_v[p], out_hbm.at[pl.ds(out_base, CHUNK // 2)],
            ssems[p]).wait()

    def compute(g, p):
        gath, outb = gath_v[p], outb_v[p]
        p0 = lax.rem(g * CHUNK, SEQ)
        iota = lax.iota(jnp.int32, LANES)

        @pl.loop(0, CHUNK // LANES)
        def _group(rg):
            r0 = rg * LANES
            xv = idx_v[g, pl.ds(r0, LANES)]
            offv = (xv & 1) << 6
            for j in range(LANES):
                r = r0 + j
                offs = lax.gather(
                    offv, jnp.full((LANES, 1), j, jnp.int32),
                    dimension_numbers=lax.GatherDimensionNumbers(
                        offset_dims=(), collapsed_slice_dims=(0,),
                        start_index_map=(0,)),
                    slice_sizes=(1,),
                    mode=lax.GatherScatterMode.PROMISE_IN_BOUNDS)
                rows = jnp.full((LANES,), r, jnp.int32)
                pr = p0 + r
                pr = jnp.where(pr >= SEQ, pr - SEQ, pr)
                pcol = (pr & 1) << 6
                orow = rg * (LANES // 2) + j // 2
                for c in range(0, D_MODEL, LANES):
                    val = plsc.load_gather(gath, [rows, offs + (c + iota)])
                    outb[orow, pl.ds(((j & 1) << 6) + c, LANES)] = (
                        val * 8.0 + pe_v[pr >> 1, pl.ds(pcol + c, LANES)])

    @pl.loop(0, N_CHUNKS)
    def _chunk(g):
        prep_chunk(g, 0)
        start_gather(0)
        wait_gather(0)
        compute(g, 0)
        pltpu.sync_copy(
            outb_v[0],
            out_hbm.at[pl.ds(out_base + g * (CHUNK // 2), CHUNK // 2)])


def kernel(x, table):
    table2 = table.reshape(table.shape[0] // 2, 128)
    idx = x.reshape(NW, N_CHUNKS, CHUNK)
    pe = _positional_encoding(SEQ, D_MODEL).reshape(SEQ // 2, 128)

    mesh = plsc.VectorSubcoreMesh(
        core_axis_name="c", subcore_axis_name="s",
        num_cores=NUM_CORES, num_subcores=NUM_SUBCORES,
    )
    k = pl.kernel(
        _sc_body,
        out_type=jax.ShapeDtypeStruct((TOTAL_ROWS // 2, 128), jnp.float32),
        mesh=mesh,
        scratch_types=[
            pltpu.VMEM((N_CHUNKS, CHUNK), jnp.int32),
            pltpu.VMEM((SEQ // 2, 128), jnp.float32),
            [pltpu.VMEM((CHUNK,), jnp.int32) for _ in range(NBUF)],
            [pltpu.VMEM((CHUNK, 128), jnp.float32) for _ in range(NBUF)],
            [pltpu.VMEM((CHUNK // 2, 128), jnp.float32) for _ in range(NBUF)],
            [pltpu.SemaphoreType.DMA for _ in range(NBUF)],
            [pltpu.SemaphoreType.DMA for _ in range(NBUF)],
        ],
        compiler_params=pltpu.CompilerParams(needs_layout_passes=False),
    )
    out = k(table2, idx, pe)
    return out.reshape(BATCH, SEQ, D_MODEL)


# speedup vs baseline: 1.1353x; 1.1353x over previous
import jax
import jax.numpy as jnp
from jax import lax
from jax.experimental import pallas as pl
from jax.experimental.pallas import tpu as pltpu
from jax.experimental.pallas import tpu_sc as plsc

D_MODEL = 64
SEQ = 200
BATCH = 4096
LANES = 16
NUM_CORES = 2
NUM_SUBCORES = 16
NW = NUM_CORES * NUM_SUBCORES
TOTAL_ROWS = BATCH * SEQ
ROWS_PER_W = TOTAL_ROWS // NW
CHUNK = 64
N_CHUNKS = ROWS_PER_W // CHUNK
IDX_COLS = 128
NBUF = 4


def _positional_encoding(length, depth):
    half = depth // 2
    positions = jnp.arange(length, dtype=jnp.float32)[:, None]
    depths = jnp.arange(half, dtype=jnp.float32)[None, :]
    angle_rates = 1.0 / (10000.0 ** depths)
    angle_rads = positions * angle_rates
    return jnp.concatenate([jnp.sin(angle_rads), jnp.cos(angle_rads)], axis=-1)


def _sc_body(table_hbm, idx_hbm, pe_hbm, out_hbm,
             idx_v, pe_v, idx2_v, gath_v, outb_v, gsems, ssems):
    wid = lax.axis_index("s") * NUM_CORES + lax.axis_index("c")
    pltpu.sync_copy(idx_hbm.at[wid], idx_v)
    pltpu.sync_copy(pe_hbm, pe_v)
    out_base = wid * (ROWS_PER_W // 2)

    def idx_block(g, c):
        return idx_v[g >> 1, pl.ds((g & 1) * CHUNK + c, LANES)]

    def prep_chunk(g, p):
        for c in range(0, CHUNK, LANES):
            idx2_v[p][pl.ds(c, LANES)] = idx_block(g, c) >> 1

    def start_gather(p):
        pltpu.async_copy(table_hbm.at[idx2_v[p]], gath_v[p], gsems[p])

    def wait_gather(p):
        pltpu.make_async_copy(table_hbm.at[idx2_v[p]], gath_v[p],
                              gsems[p]).wait()

    def start_store(g, p):
        pltpu.async_copy(
            outb_v[p],
            out_hbm.at[pl.ds(out_base + g * (CHUNK // 2), CHUNK // 2)],
            ssems[p])

    def wait_store(p):
        pltpu.make_async_copy(
            outb_v[p], out_hbm.at[pl.ds(out_base, CHUNK // 2)],
            ssems[p]).wait()

    def compute(g, p):
        gath, outb = gath_v[p], outb_v[p]
        p0 = lax.rem(g * CHUNK, SEQ)
        iota = lax.iota(jnp.int32, LANES)

        @pl.loop(0, CHUNK // LANES)
        def _group(rg):
            r0 = rg * LANES
            offv = (idx_block(g, r0) & 1) << 6
            for j in range(LANES):
                r = r0 + j
                offs = lax.gather(
                    offv, jnp.full((LANES, 1), j, jnp.int32),
                    dimension_numbers=lax.GatherDimensionNumbers(
                        offset_dims=(), collapsed_slice_dims=(0,),
                        start_index_map=(0,)),
                    slice_sizes=(1,),
                    mode=lax.GatherScatterMode.PROMISE_IN_BOUNDS)
                rows = jnp.full((LANES,), r, jnp.int32)
                pr = p0 + r
                pr = jnp.where(pr >= SEQ, pr - SEQ, pr)
                pcol = (pr & 1) << 6
                orow = rg * (LANES // 2) + j // 2
                for c in range(0, D_MODEL, LANES):
                    val = plsc.load_gather(gath, [rows, offs + (c + iota)])
                    outb[orow, pl.ds(((j & 1) << 6) + c, LANES)] = (
                        val * 8.0 + pe_v[pr >> 1, pl.ds(pcol + c, LANES)])

    prep_chunk(0, 0)
    start_gather(0)

    @pl.loop(0, N_CHUNKS // NBUF)
    def _ring(h):
        for p in range(NBUF):
            g = h * NBUF + p
            wait_gather(p)
            nxt = (p + 1) % NBUF
            @pl.when(g >= NBUF - 1)
            def _():
                wait_store(nxt)

            @pl.when(g + 1 < N_CHUNKS)
            def _():
                prep_chunk(g + 1, nxt)
                start_gather(nxt)

            compute(g, p)
            start_store(g, p)

    for p in range(1, NBUF):
        wait_store(p)


def kernel(x, table):
    table2 = table.reshape(table.shape[0] // 2, 128)
    idx = x.reshape(NW, N_CHUNKS // 2, IDX_COLS)
    pe = _positional_encoding(SEQ, D_MODEL).reshape(SEQ // 2, 128)

    mesh = plsc.VectorSubcoreMesh(
        core_axis_name="c", subcore_axis_name="s",
        num_cores=NUM_CORES, num_subcores=NUM_SUBCORES,
    )
    k = pl.kernel(
        _sc_body,
        out_type=jax.ShapeDtypeStruct((TOTAL_ROWS // 2, 128), jnp.float32),
        mesh=mesh,
        scratch_types=[
            pltpu.VMEM((N_CHUNKS // 2, IDX_COLS), jnp.int32),
            pltpu.VMEM((SEQ // 2, 128), jnp.float32),
            [pltpu.VMEM((CHUNK,), jnp.int32) for _ in range(NBUF)],
            [pltpu.VMEM((CHUNK, 128), jnp.float32) for _ in range(NBUF)],
            [pltpu.VMEM((CHUNK // 2, 128), jnp.float32) for _ in range(NBUF)],
            [pltpu.SemaphoreType.DMA for _ in range(NBUF)],
            [pltpu.SemaphoreType.DMA for _ in range(NBUF)],
        ],
        compiler_params=pltpu.CompilerParams(needs_layout_passes=False),
    )
    out = k(table2, idx, pe)
    return out.reshape(BATCH, SEQ, D_MODEL)
